# baseline (speedup 1.0000x reference)
import functools
import jax
import jax.numpy as jnp
from jax import lax
from jax.experimental import pallas as pl
from jax.experimental.pallas import tpu as pltpu
from jax.experimental.pallas import tpu_sc as plsc

VOCAB = 1000000
D = 64
DP = 128
NC, NS = 2, 16
NW = NC * NS
CHUNK = 256
NBUF = 3

TVB = 2048
TGRID = -(-VOCAB // TVB)


def _tc_transpose_body(xt_ref, out_ref):
    out_ref[:, 0:D] = xt_ref[...].T


def _gather_body(idx_hbm, table_hbm, out_hbm, idx_v, *scratch,
                 b_per_w, nchunk):
    bufs = scratch[:NBUF]
    sem_g = scratch[NBUF:2 * NBUF]
    sem_s = scratch[2 * NBUF:3 * NBUF]

    wid = lax.axis_index("s") * NC + lax.axis_index("c")
    base = wid * b_per_w
    pltpu.sync_copy(idx_hbm.at[pl.ds(base, b_per_w)], idx_v)

    gathers = {}
    stores = {}
    store_waited = set()

    def start_gather(g):
        b = g % NBUF
        gathers[g] = pltpu.async_copy(
            table_hbm.at[idx_v.at[pl.ds(g * CHUNK, CHUNK)]], bufs[b],
            sem_g[b])

    for g in range(min(NBUF, nchunk)):
        start_gather(g)

    for g in range(nchunk):
        b = g % NBUF
        gathers[g].wait()
        stores[g] = pltpu.async_copy(
            bufs[b], out_hbm.at[pl.ds(base + g * CHUNK, CHUNK)], sem_s[b])
        t = g - 1 + NBUF
        if g >= 1 and t < nchunk:
            stores[g - 1].wait()
            store_waited.add(g - 1)
            start_gather(t)

    for g in range(nchunk):
        if g not in store_waited:
            stores[g].wait()


def kernel(indices, X):
    batch, n_fields = indices.shape
    b_total = batch * n_fields
    assert b_total % (8 * NW) == 0
    b_per_w = b_total // NW
    assert b_per_w % CHUNK == 0
    nchunk = b_per_w // CHUNK

    flat_idx = indices.reshape(b_total).astype(jnp.int32)

    transpose = pl.pallas_call(
        _tc_transpose_body,
        grid=(TGRID,),
        in_specs=[pl.BlockSpec((D, TVB), lambda i: (0, i))],
        out_specs=pl.BlockSpec((TVB, DP), lambda i: (i, 0)),
        out_shape=jax.ShapeDtypeStruct((VOCAB, DP), jnp.float32),
    )
    x_wide = transpose(X.T)

    mesh = plsc.VectorSubcoreMesh(core_axis_name="c", subcore_axis_name="s")
    gather = pl.kernel(
        functools.partial(_gather_body, b_per_w=b_per_w, nchunk=nchunk),
        mesh=mesh,
        out_type=jax.ShapeDtypeStruct((b_total, DP), jnp.float32),
        scratch_types=(
            [pltpu.VMEM((b_per_w,), jnp.int32)]
            + [pltpu.VMEM((CHUNK, DP), jnp.float32) for _ in range(NBUF)]
            + [pltpu.SemaphoreType.DMA for _ in range(2 * NBUF)]
        ),
        compiler_params=pltpu.CompilerParams(
            use_tc_tiling_on_sc=True, needs_layout_passes=False),
    )
    out = gather(flat_idx, x_wide)
    return out[:, :D].reshape(batch, n_fields, D)


# speedup vs baseline: 1.2020x; 1.2020x over previous
import functools
import jax
import jax.numpy as jnp
from jax import lax
from jax.experimental import pallas as pl
from jax.experimental.pallas import tpu as pltpu
from jax.experimental.pallas import tpu_sc as plsc

VOCAB = 1000000
D = 64
DP = 128
NC, NS = 2, 16
NW = NC * NS
CHUNK = 416
NBUF = 2

TVB = 4096
TGRID = -(-VOCAB // TVB)


def _tc_transpose_body(xt_ref, out_ref):
    out_ref[:, 0:D] = xt_ref[...].T


def _gather_body(idx_hbm, table_hbm, out_hbm, idx_v, *scratch,
                 b_per_w, nchunk):
    bufs = scratch[:NBUF]
    sem_g = scratch[NBUF:2 * NBUF]
    sem_s = scratch[2 * NBUF:3 * NBUF]

    wid = lax.axis_index("s") * NC + lax.axis_index("c")
    base = wid * b_per_w
    pltpu.sync_copy(idx_hbm.at[pl.ds(base, b_per_w)], idx_v)

    gathers = {}
    stores = {}
    store_waited = set()

    def start_gather(g):
        b = g % NBUF
        gathers[g] = pltpu.async_copy(
            table_hbm.at[idx_v.at[pl.ds(g * CHUNK, CHUNK)]], bufs[b],
            sem_g[b])

    for g in range(min(NBUF, nchunk)):
        start_gather(g)

    for g in range(nchunk):
        b = g % NBUF
        gathers[g].wait()
        stores[g] = pltpu.async_copy(
            bufs[b].at[:, pl.ds(0, D)],
            out_hbm.at[pl.ds(base + g * CHUNK, CHUNK)], sem_s[b])
        t = g - 1 + NBUF
        if g >= 1 and t < nchunk:
            stores[g - 1].wait()
            store_waited.add(g - 1)
            start_gather(t)

    for g in range(nchunk):
        if g not in store_waited:
            stores[g].wait()


def kernel(indices, X):
    batch, n_fields = indices.shape
    b_total = batch * n_fields
    assert b_total % (8 * NW) == 0
    b_per_w = b_total // NW
    assert b_per_w % CHUNK == 0
    nchunk = b_per_w // CHUNK

    flat_idx = indices.reshape(b_total).astype(jnp.int32)

    transpose = pl.pallas_call(
        _tc_transpose_body,
        grid=(TGRID,),
        in_specs=[pl.BlockSpec((D, TVB), lambda i: (0, i))],
        out_specs=pl.BlockSpec((TVB, DP), lambda i: (i, 0)),
        out_shape=jax.ShapeDtypeStruct((VOCAB, DP), jnp.float32),
    )
    x_wide = transpose(X.T)

    mesh = plsc.VectorSubcoreMesh(core_axis_name="c", subcore_axis_name="s")
    gather = pl.kernel(
        functools.partial(_gather_body, b_per_w=b_per_w, nchunk=nchunk),
        mesh=mesh,
        out_type=jax.ShapeDtypeStruct((b_total, D), jnp.float32),
        scratch_types=(
            [pltpu.VMEM((b_per_w,), jnp.int32)]
            + [pltpu.VMEM((CHUNK, DP), jnp.float32) for _ in range(NBUF)]
            + [pltpu.SemaphoreType.DMA for _ in range(2 * NBUF)]
        ),
        compiler_params=pltpu.CompilerParams(use_tc_tiling_on_sc=False),
    )
    out = gather(flat_idx, x_wide)
    return out.reshape(batch, n_fields, D)


# speedup vs baseline: 1.3227x; 1.1003x over previous
import functools
import jax
import jax.numpy as jnp
from jax import lax
from jax.experimental import pallas as pl
from jax.experimental.pallas import tpu as pltpu
from jax.experimental.pallas import tpu_sc as plsc

VOCAB = 1000000
D = 64
DP = 128
NC, NS = 2, 16
NW = NC * NS
CHUNK = 416
NBUF = 2

TVB = 8192
TGRID = -(-VOCAB // TVB)


def _tc_transpose_body(xt_ref, out_ref):
    out_ref[:, 0:D] = xt_ref[...].T


def _gather_body(idx_hbm, table_hbm, out_hbm, idx_v, *scratch,
                 b_per_w, nchunk):
    bufs = scratch[:NBUF]
    sem_g = scratch[NBUF:2 * NBUF]
    sem_s = scratch[2 * NBUF:3 * NBUF]

    wid = lax.axis_index("s") * NC + lax.axis_index("c")
    base = wid * b_per_w
    pltpu.sync_copy(idx_hbm.at[pl.ds(base, b_per_w)], idx_v)

    gathers = {}
    stores = {}
    store_waited = set()

    def start_gather(g):
        b = g % NBUF
        gathers[g] = pltpu.async_copy(
            table_hbm.at[idx_v.at[pl.ds(g * CHUNK, CHUNK)]], bufs[b],
            sem_g[b])

    for g in range(min(NBUF, nchunk)):
        start_gather(g)

    for g in range(nchunk):
        b = g % NBUF
        gathers[g].wait()
        stores[g] = pltpu.async_copy(
            bufs[b].at[:, pl.ds(0, D)],
            out_hbm.at[pl.ds(base + g * CHUNK, CHUNK)], sem_s[b])
        t = g - 1 + NBUF
        if g >= 1 and t < nchunk:
            stores[g - 1].wait()
            store_waited.add(g - 1)
            start_gather(t)

    for g in range(nchunk):
        if g not in store_waited:
            stores[g].wait()


def kernel(indices, X):
    batch, n_fields = indices.shape
    b_total = batch * n_fields
    assert b_total % (8 * NW) == 0
    b_per_w = b_total // NW
    assert b_per_w % CHUNK == 0
    nchunk = b_per_w // CHUNK

    flat_idx = indices.reshape(b_total).astype(jnp.int32)

    transpose = pl.pallas_call(
        _tc_transpose_body,
        grid=(TGRID,),
        in_specs=[pl.BlockSpec((D, TVB), lambda i: (0, i))],
        out_specs=pl.BlockSpec((TVB, DP), lambda i: (i, 0)),
        out_shape=jax.ShapeDtypeStruct((VOCAB, DP), jnp.float32),
    )
    x_wide = transpose(X.T)

    mesh = plsc.VectorSubcoreMesh(core_axis_name="c", subcore_axis_name="s")
    gather = pl.kernel(
        functools.partial(_gather_body, b_per_w=b_per_w, nchunk=nchunk),
        mesh=mesh,
        out_type=jax.ShapeDtypeStruct((b_total, D), jnp.float32),
        scratch_types=(
            [pltpu.VMEM((b_per_w,), jnp.int32)]
            + [pltpu.VMEM((CHUNK, DP), jnp.float32) for _ in range(NBUF)]
            + [pltpu.SemaphoreType.DMA for _ in range(2 * NBUF)]
        ),
        compiler_params=pltpu.CompilerParams(use_tc_tiling_on_sc=False),
    )
    out = gather(flat_idx, x_wide)
    return out.reshape(batch, n_fields, D)


# speedup vs baseline: 1.3641x; 1.0314x over previous
import functools
import jax
import jax.numpy as jnp
from jax import lax
from jax.experimental import pallas as pl
from jax.experimental.pallas import tpu as pltpu
from jax.experimental.pallas import tpu_sc as plsc

VOCAB = 1000000
D = 64
DP = 128
NC, NS = 2, 16
NW = NC * NS
CHUNK = 416
NBUF = 2

TVB = 16384
TGRID = -(-VOCAB // TVB)


def _tc_transpose_body(xt_ref, out_ref):
    out_ref[:, 0:D] = xt_ref[...].T


def _gather_body(idx_hbm, table_hbm, out_hbm, idx_v, *scratch,
                 b_per_w, nchunk):
    bufs = scratch[:NBUF]
    sem_g = scratch[NBUF:2 * NBUF]
    sem_s = scratch[2 * NBUF:3 * NBUF]

    wid = lax.axis_index("s") * NC + lax.axis_index("c")
    base = wid * b_per_w
    pltpu.sync_copy(idx_hbm.at[pl.ds(base, b_per_w)], idx_v)

    gathers = {}
    stores = {}
    store_waited = set()

    def start_gather(g):
        b = g % NBUF
        gathers[g] = pltpu.async_copy(
            table_hbm.at[idx_v.at[pl.ds(g * CHUNK, CHUNK)]], bufs[b],
            sem_g[b])

    for g in range(min(NBUF, nchunk)):
        start_gather(g)

    for g in range(nchunk):
        b = g % NBUF
        gathers[g].wait()
        stores[g] = pltpu.async_copy(
            bufs[b].at[:, pl.ds(0, D)],
            out_hbm.at[pl.ds(base + g * CHUNK, CHUNK)], sem_s[b])
        t = g - 1 + NBUF
        if g >= 1 and t < nchunk:
            stores[g - 1].wait()
            store_waited.add(g - 1)
            start_gather(t)

    for g in range(nchunk):
        if g not in store_waited:
            stores[g].wait()


def kernel(indices, X):
    batch, n_fields = indices.shape
    b_total = batch * n_fields
    assert b_total % (8 * NW) == 0
    b_per_w = b_total // NW
    assert b_per_w % CHUNK == 0
    nchunk = b_per_w // CHUNK

    flat_idx = indices.reshape(b_total).astype(jnp.int32)

    transpose = pl.pallas_call(
        _tc_transpose_body,
        grid=(TGRID,),
        in_specs=[pl.BlockSpec((D, TVB), lambda i: (0, i))],
        out_specs=pl.BlockSpec((TVB, DP), lambda i: (i, 0)),
        out_shape=jax.ShapeDtypeStruct((VOCAB, DP), jnp.float32),
    )
    x_wide = transpose(X.T)

    mesh = plsc.VectorSubcoreMesh(core_axis_name="c", subcore_axis_name="s")
    gather = pl.kernel(
        functools.partial(_gather_body, b_per_w=b_per_w, nchunk=nchunk),
        mesh=mesh,
        out_type=jax.ShapeDtypeStruct((b_total, D), jnp.float32),
        scratch_types=(
            [pltpu.VMEM((b_per_w,), jnp.int32)]
            + [pltpu.VMEM((CHUNK, DP), jnp.float32) for _ in range(NBUF)]
            + [pltpu.SemaphoreType.DMA for _ in range(2 * NBUF)]
        ),
        compiler_params=pltpu.CompilerParams(use_tc_tiling_on_sc=False),
    )
    out = gather(flat_idx, x_wide)
    return out.reshape(batch, n_fields, D)


# speedup vs baseline: 1.3730x; 1.0065x over previous
import functools
import jax
import jax.numpy as jnp
from jax import lax
from jax.experimental import pallas as pl
from jax.experimental.pallas import tpu as pltpu
from jax.experimental.pallas import tpu_sc as plsc

VOCAB = 1000000
D = 64
DP = 128
NC, NS = 2, 16
NW = NC * NS
CHUNK = 416
NBUF = 2

TVB = 32768
TGRID = -(-VOCAB // TVB)


def _tc_transpose_body(xt_ref, out_ref):
    out_ref[:, 0:D] = xt_ref[...].T


def _gather_body(idx_hbm, table_hbm, out_hbm, idx_v, *scratch,
                 b_per_w, nchunk):
    bufs = scratch[:NBUF]
    sem_g = scratch[NBUF:2 * NBUF]
    sem_s = scratch[2 * NBUF:3 * NBUF]

    wid = lax.axis_index("s") * NC + lax.axis_index("c")
    base = wid * b_per_w
    pltpu.sync_copy(idx_hbm.at[pl.ds(base, b_per_w)], idx_v)

    gathers = {}
    stores = {}
    store_waited = set()

    def start_gather(g):
        b = g % NBUF
        gathers[g] = pltpu.async_copy(
            table_hbm.at[idx_v.at[pl.ds(g * CHUNK, CHUNK)]], bufs[b],
            sem_g[b])

    for g in range(min(NBUF, nchunk)):
        start_gather(g)

    for g in range(nchunk):
        b = g % NBUF
        gathers[g].wait()
        stores[g] = pltpu.async_copy(
            bufs[b].at[:, pl.ds(0, D)],
            out_hbm.at[pl.ds(base + g * CHUNK, CHUNK)], sem_s[b])
        t = g - 1 + NBUF
        if g >= 1 and t < nchunk:
            stores[g - 1].wait()
            store_waited.add(g - 1)
            start_gather(t)

    for g in range(nchunk):
        if g not in store_waited:
            stores[g].wait()


def kernel(indices, X):
    batch, n_fields = indices.shape
    b_total = batch * n_fields
    assert b_total % (8 * NW) == 0
    b_per_w = b_total // NW
    assert b_per_w % CHUNK == 0
    nchunk = b_per_w // CHUNK

    flat_idx = indices.reshape(b_total).astype(jnp.int32)

    transpose = pl.pallas_call(
        _tc_transpose_body,
        grid=(TGRID,),
        in_specs=[pl.BlockSpec((D, TVB), lambda i: (0, i))],
        out_specs=pl.BlockSpec((TVB, DP), lambda i: (i, 0)),
        out_shape=jax.ShapeDtypeStruct((VOCAB, DP), jnp.float32),
    )
    x_wide = transpose(X.T)

    mesh = plsc.VectorSubcoreMesh(core_axis_name="c", subcore_axis_name="s")
    gather = pl.kernel(
        functools.partial(_gather_body, b_per_w=b_per_w, nchunk=nchunk),
        mesh=mesh,
        out_type=jax.ShapeDtypeStruct((b_total, D), jnp.float32),
        scratch_types=(
            [pltpu.VMEM((b_per_w,), jnp.int32)]
            + [pltpu.VMEM((CHUNK, DP), jnp.float32) for _ in range(NBUF)]
            + [pltpu.SemaphoreType.DMA for _ in range(2 * NBUF)]
        ),
        compiler_params=pltpu.CompilerParams(use_tc_tiling_on_sc=False),
    )
    out = gather(flat_idx, x_wide)
    return out.reshape(batch, n_fields, D)
